# baseline (speedup 1.0000x reference)
import functools

import jax
import jax.numpy as jnp
from jax import lax
from jax.experimental import pallas as pl
from jax.experimental.pallas import tpu as pltpu
from jax.experimental.pallas import tpu_sc as plsc

_N = 10000
_E = 320000
_H = 128
_G = 128
_OUT = 64
_L = 4

_NC = 2
_NS = 16
_NW = _NC * _NS
_EGRP = 128
_GROUPS_PER_W = 80
_E_PAD = _GROUPS_PER_W * _NW * _EGRP
_DCH = 16
_NDCH = _GROUPS_PER_W // _DCH
_NPAD = 10112
_ZROWS = _NPAD // _NS


def _sc_aggregate(h, src_g, dst_g, zeros_init):
  mesh = plsc.VectorSubcoreMesh(
      core_axis_name="c", subcore_axis_name="s",
      num_cores=_NC, num_subcores=_NS)

  @functools.partial(
      pl.kernel,
      out_type=jax.ShapeDtypeStruct((_NC, _NPAD, _H), jnp.float32),
      mesh=mesh,
      scratch_types=[
          pltpu.VMEM((_GROUPS_PER_W, _EGRP), jnp.int32),
          pltpu.VMEM((_DCH, _EGRP), jnp.int32),
          pltpu.VMEM((_DCH, _EGRP), jnp.int32),
          pltpu.VMEM((_EGRP, _H), jnp.float32),
          pltpu.VMEM((_EGRP, _H), jnp.float32),
          pltpu.VMEM_SHARED((_NPAD, _H), jnp.float32),
          pltpu.SemaphoreType.DMA,
          pltpu.SemaphoreType.DMA,
          pltpu.SemaphoreType.DMA,
          pltpu.SemaphoreType.DMA,
      ],
  )
  def agg_kernel(h_hbm, src_hbm, dst_hbm, z_hbm, out_hbm,
                 sidx, didx0, didx1, rows0, rows1, agg_sh,
                 gsem0, gsem1, isem0, isem1):
    c = lax.axis_index("c")
    s = lax.axis_index("s")
    w = c * _NS + s
    gsems = (gsem0, gsem1)
    isems = (isem0, isem1)
    didxs = (didx0, didx1)
    bufs = (rows0, rows1)
    pltpu.sync_copy(z_hbm.at[pl.ds(s * _ZROWS, _ZROWS)],
                    agg_sh.at[pl.ds(s * _ZROWS, _ZROWS)])
    pltpu.sync_copy(src_hbm.at[w], sidx)
    pltpu.sync_copy(dst_hbm.at[w, pl.ds(0, _DCH)], didx0)
    plsc.subcore_barrier()
    pltpu.async_copy(dst_hbm.at[w, pl.ds(_DCH, _DCH)], didx1, isems[1])

    for ch in range(_NDCH):
      sl = ch % 2
      if ch > 0:
        pltpu.make_async_copy(dst_hbm.at[w, pl.ds(ch * _DCH, _DCH)],
                              didxs[sl], isems[sl]).wait()
      if ch + 1 < _NDCH:
        nsl = (ch + 1) % 2
        pltpu.async_copy(dst_hbm.at[w, pl.ds((ch + 1) * _DCH, _DCH)],
                         didxs[nsl], isems[nsl])

      def body(i, carry, ch=ch, sl=sl):
        for b in range(2):
          jl = i * 2 + b
          j = ch * _DCH + jl
          pltpu.async_copy(h_hbm.at[sidx.at[j]], bufs[b], gsems[b]).wait()
          pltpu.sync_copy(bufs[b], agg_sh.at[didxs[sl].at[jl]], add=True)
        return carry

      lax.fori_loop(0, _DCH // 2, body, 0)
    plsc.subcore_barrier()
    pltpu.sync_copy(agg_sh.at[pl.ds(s * _ZROWS, _ZROWS)],
                    out_hbm.at[c, pl.ds(s * _ZROWS, _ZROWS)])

  return agg_kernel(h, src_g, dst_g, zeros_init)


def _layer_body(h_ref, agg_ref, w1, b1, g1, be1, w2, b2, g2, be2, out_ref):
  z = h_ref[...] + agg_ref[0, :_N] + agg_ref[1, :_N]
  y = jnp.dot(z, w1[...], preferred_element_type=jnp.float32) + b1[...]
  mu = jnp.mean(y, axis=0, keepdims=True)
  yc = y - mu
  var = jnp.mean(yc * yc, axis=0, keepdims=True)
  m = jnp.maximum(g1[...] * yc / jnp.sqrt(var + 1e-5) + be1[...], 0.0)
  y2 = jnp.dot(m, w2[...], preferred_element_type=jnp.float32) + b2[...]
  mu2 = jnp.mean(y2, axis=0, keepdims=True)
  yc2 = y2 - mu2
  var2 = jnp.mean(yc2 * yc2, axis=0, keepdims=True)
  out_ref[...] = jnp.maximum(
      g2[...] * yc2 / jnp.sqrt(var2 + 1e-5) + be2[...], 0.0)


def _tc_layer(h, agg, p):
  return pl.pallas_call(
      _layer_body,
      out_shape=jax.ShapeDtypeStruct((_N, _H), jnp.float32),
  )(h, agg, p['W1'], p['b1'], p['g1'], p['be1'],
    p['W2'], p['b2'], p['g'], p['be'])


def _pool_body(batch_ref, o0, o1, o2, o3, o4,
               w0, w1, w2, w3, w4, b0, b1, b2, b3, b4, out_ref):
  onehot = (batch_ref[...] ==
            lax.broadcasted_iota(jnp.int32, (_N, _G), 1)).astype(jnp.float32)
  dn = (((0,), (0,)), ((), ()))
  acc = None
  for o_ref, w_ref in ((o0, w0), (o1, w1), (o2, w2), (o3, w3), (o4, w4)):
    pooled = lax.dot_general(onehot, o_ref[...], dimension_numbers=dn,
                             preferred_element_type=jnp.float32)
    y = jnp.dot(pooled, w_ref[...], preferred_element_type=jnp.float32)
    acc = y if acc is None else acc + y
  bias = b0[...] + b1[...] + b2[...] + b3[...] + b4[...]
  out_ref[...] = acc + bias[None, :]


def _tc_pool(batch2d, outs, fcs):
  args = [batch2d] + outs + [f['W'] for f in fcs] + [f['b'] for f in fcs]
  return pl.pallas_call(
      _pool_body,
      out_shape=jax.ShapeDtypeStruct((_G, _OUT), jnp.float32),
  )(*args)


def kernel(x, edge_index, batch, params):
  src = edge_index[0]
  dst = edge_index[1]
  pad = _E_PAD - _E
  src_g = jnp.concatenate(
      [src, jnp.zeros((pad,), jnp.int32)]).reshape(_NW, _GROUPS_PER_W, _EGRP)
  dst_g = jnp.concatenate(
      [dst, jnp.full((pad,), _N, jnp.int32)]).reshape(_NW, _GROUPS_PER_W, _EGRP)
  zeros_init = jnp.zeros((_NPAD, _H), jnp.float32)
  batch2d = batch.reshape(_N, 1)

  outs = [x]
  h = x
  for i in range(_L):
    agg = _sc_aggregate(h, src_g, dst_g, zeros_init)
    h = _tc_layer(h, agg, params['conv%d' % i])
    outs.append(h)
  return _tc_pool(batch2d, outs, params['fcs'])


# speedup vs baseline: 1.5383x; 1.5383x over previous
import functools

import jax
import jax.numpy as jnp
from jax import lax
from jax.experimental import pallas as pl
from jax.experimental.pallas import tpu as pltpu
from jax.experimental.pallas import tpu_sc as plsc

_N = 10000
_E = 320000
_H = 128
_G = 128
_OUT = 64
_L = 4

_NC = 2
_NS = 16
_NW = _NC * _NS
_EGRP = 128
_GROUPS_PER_W = 79
_E_PAD = _GROUPS_PER_W * _NW * _EGRP
_NPAD = 10112
_ZROWS = _NPAD // _NS


def _sc_aggregate(h, src_g, dst_g, zeros_init):
  mesh = plsc.VectorSubcoreMesh(
      core_axis_name="c", subcore_axis_name="s",
      num_cores=_NC, num_subcores=_NS)

  @functools.partial(
      pl.kernel,
      out_type=jax.ShapeDtypeStruct((_NC, _NPAD, _H), jnp.float32),
      mesh=mesh,
      scratch_types=[
          pltpu.VMEM((_GROUPS_PER_W, _EGRP), jnp.int32),
          pltpu.VMEM((_GROUPS_PER_W, _EGRP), jnp.int32),
          pltpu.VMEM((_EGRP, _H), jnp.float32),
          pltpu.VMEM_SHARED((_NPAD, _H), jnp.float32),
          pltpu.SemaphoreType.DMA,
      ],
  )
  def agg_kernel(h_hbm, src_hbm, dst_hbm, z_hbm, out_hbm,
                 sidx, didx, rows, agg_sh, sem):
    c = lax.axis_index("c")
    s = lax.axis_index("s")
    w = c * _NS + s
    pltpu.sync_copy(z_hbm.at[pl.ds(s * _ZROWS, _ZROWS)],
                    agg_sh.at[pl.ds(s * _ZROWS, _ZROWS)])
    pltpu.sync_copy(src_hbm.at[w], sidx)
    pltpu.sync_copy(dst_hbm.at[w], didx)
    plsc.subcore_barrier()

    def body(j, carry):
      pltpu.async_copy(h_hbm.at[sidx.at[j]], rows, sem).wait()
      pltpu.sync_copy(rows, agg_sh.at[didx.at[j]], add=True)
      return carry

    lax.fori_loop(0, _GROUPS_PER_W, body, 0)
    plsc.subcore_barrier()
    pltpu.sync_copy(agg_sh.at[pl.ds(s * _ZROWS, _ZROWS)],
                    out_hbm.at[c, pl.ds(s * _ZROWS, _ZROWS)])

  return agg_kernel(h, src_g, dst_g, zeros_init)


def _layer_body(h_ref, agg_ref, w1, b1, g1, be1, w2, b2, g2, be2, out_ref):
  z = h_ref[...] + agg_ref[0, :_N] + agg_ref[1, :_N]
  y = jnp.dot(z, w1[...], preferred_element_type=jnp.float32) + b1[...]
  mu = jnp.mean(y, axis=0, keepdims=True)
  yc = y - mu
  var = jnp.mean(yc * yc, axis=0, keepdims=True)
  m = jnp.maximum(g1[...] * yc / jnp.sqrt(var + 1e-5) + be1[...], 0.0)
  y2 = jnp.dot(m, w2[...], preferred_element_type=jnp.float32) + b2[...]
  mu2 = jnp.mean(y2, axis=0, keepdims=True)
  yc2 = y2 - mu2
  var2 = jnp.mean(yc2 * yc2, axis=0, keepdims=True)
  out_ref[...] = jnp.maximum(
      g2[...] * yc2 / jnp.sqrt(var2 + 1e-5) + be2[...], 0.0)


def _tc_layer(h, agg, p):
  return pl.pallas_call(
      _layer_body,
      out_shape=jax.ShapeDtypeStruct((_N, _H), jnp.float32),
  )(h, agg, p['W1'], p['b1'], p['g1'], p['be1'],
    p['W2'], p['b2'], p['g'], p['be'])


def _pool_body(batch_ref, o0, o1, o2, o3, o4,
               w0, w1, w2, w3, w4, b0, b1, b2, b3, b4, out_ref):
  onehot = (batch_ref[...] ==
            lax.broadcasted_iota(jnp.int32, (_N, _G), 1)).astype(jnp.float32)
  dn = (((0,), (0,)), ((), ()))
  acc = None
  for o_ref, w_ref in ((o0, w0), (o1, w1), (o2, w2), (o3, w3), (o4, w4)):
    pooled = lax.dot_general(onehot, o_ref[...], dimension_numbers=dn,
                             preferred_element_type=jnp.float32)
    y = jnp.dot(pooled, w_ref[...], preferred_element_type=jnp.float32)
    acc = y if acc is None else acc + y
  bias = b0[...] + b1[...] + b2[...] + b3[...] + b4[...]
  out_ref[...] = acc + bias[None, :]


def _tc_pool(batch2d, outs, fcs):
  args = [batch2d] + outs + [f['W'] for f in fcs] + [f['b'] for f in fcs]
  return pl.pallas_call(
      _pool_body,
      out_shape=jax.ShapeDtypeStruct((_G, _OUT), jnp.float32),
  )(*args)


def kernel(x, edge_index, batch, params):
  src = edge_index[0]
  dst = edge_index[1]
  pad = _E_PAD - _E
  src_g = jnp.concatenate(
      [src, jnp.zeros((pad,), jnp.int32)]).reshape(_NW, _GROUPS_PER_W, _EGRP)
  dst_g = jnp.concatenate(
      [dst, jnp.full((pad,), _N, jnp.int32)]).reshape(_NW, _GROUPS_PER_W, _EGRP)
  zeros_init = jnp.zeros((_NPAD, _H), jnp.float32)
  batch2d = batch.reshape(_N, 1)

  outs = [x]
  h = x
  for i in range(_L):
    agg = _sc_aggregate(h, src_g, dst_g, zeros_init)
    h = _tc_layer(h, agg, params['conv%d' % i])
    outs.append(h)
  return _tc_pool(batch2d, outs, params['fcs'])
